# Initial kernel scaffold; baseline (speedup 1.0000x reference)
import jax
import jax.numpy as jnp
from jax.experimental import pallas as pl


def kernel(x_nodes, x_edges, edge_index, batch, pbc, params):
    raise NotImplementedError("write your pallas kernel here")


# speedup vs baseline: 3.2868x; 3.2868x over previous
import functools

import jax
import jax.numpy as jnp
from jax import lax
from jax.experimental import pallas as pl
from jax.experimental.pallas import tpu as pltpu
from jax.experimental.pallas import tpu_sc as plsc

N = 10000
E = 320000
H = 128
G = 64
NC, NS = 2, 16
NW = NC * NS
EPW = E // NW
CH = 80
NCH = EPW // CH
NPS = 624
NTAIL = N - NS * NPS
EBLK = 8000
NEB = E // EBLK

_f32 = jnp.float32
_i32 = jnp.int32


def _silu(x):
    return x * (1.0 / (1.0 + jnp.exp(-x)))


def _mlp2_body(x_ref, w1_ref, b1_ref, w2_ref, b2_ref, o_ref, *, act_last):
    h = _silu(jnp.dot(x_ref[...], w1_ref[...], preferred_element_type=_f32)
              + b1_ref[...])
    o = jnp.dot(h, w2_ref[...], preferred_element_type=_f32) + b2_ref[...]
    o_ref[...] = _silu(o) if act_last else o


def _ab_body(x_ref, wa_ref, wb_ref, a_ref, b_ref):
    x = x_ref[...]
    a_ref[...] = jnp.dot(x, wa_ref[...], preferred_element_type=_f32)
    b_ref[...] = jnp.dot(x, wb_ref[...], preferred_element_type=_f32)


def _edge1_body(xr_ref, gs_ref, gr_ref, we1, be1, we2, be2, w1c, b1, w2, b2,
                o_ref):
    t = _silu(jnp.dot(xr_ref[...], we1[...], preferred_element_type=_f32)
              + be1[...])
    xe0 = jnp.dot(t, we2[...], preferred_element_type=_f32) + be2[...]
    h = _silu(jnp.dot(xe0, w1c[...], preferred_element_type=_f32)
              + gs_ref[...] + gr_ref[...] + b1[...])
    o = jnp.dot(h, w2[...], preferred_element_type=_f32) + b2[...]
    o_ref[...] = _silu(o)


def _edge_mid_body(xe_ref, gs_ref, gr_ref, w1c, b1, w2, b2, o_ref):
    h = _silu(jnp.dot(xe_ref[...], w1c[...], preferred_element_type=_f32)
              + gs_ref[...] + gr_ref[...] + b1[...])
    o = jnp.dot(h, w2[...], preferred_element_type=_f32) + b2[...]
    o_ref[...] = _silu(o)


def _edge4_body(xe_ref, gs_ref, gr_ref, w1c, b1, w2, b2, wr1, br1, wr2, br2,
                xe4_ref, xo_ref, ep_ref):
    h = _silu(jnp.dot(xe_ref[...], w1c[...], preferred_element_type=_f32)
              + gs_ref[...] + gr_ref[...] + b1[...])
    xe4 = _silu(jnp.dot(h, w2[...], preferred_element_type=_f32) + b2[...])
    xe4_ref[...] = xe4
    r = _silu(jnp.dot(xe4, wr1[...], preferred_element_type=_f32) + br1[...])
    o = jnp.dot(r, wr2[...], preferred_element_type=_f32) + br2[...]
    xo_ref[...] = o
    ep_ref[...] = (1.0 + jnp.sum(o * o, axis=1, keepdims=True)) * 0.25


def _node_body(xn_ref, p0_ref, p1_ref, w3a, w3b, b3, w4, b4, o_ref):
    aggr = p0_ref[...] + p1_ref[...]
    h = _silu(jnp.dot(xn_ref[...], w3a[...], preferred_element_type=_f32)
              + jnp.dot(aggr, w3b[...], preferred_element_type=_f32)
              + b3[...])
    o_ref[...] = jnp.dot(h, w4[...], preferred_element_type=_f32) + b4[...]


def _nro_body(xn_ref, wr1, br1, wr2, br2, xo_ref, np_ref):
    r = _silu(jnp.dot(xn_ref[...], wr1[...], preferred_element_type=_f32)
              + br1[...])
    o = jnp.dot(r, wr2[...], preferred_element_type=_f32) + br2[...]
    xo_ref[...] = o
    np_ref[...] = (1.0 + jnp.sum(o * o, axis=1, keepdims=True)) * 0.5


def _starts_body(b_ref, s_ref, e_ref):
    blk = b_ref.shape[0]
    gi = lax.broadcasted_iota(_i32, (blk, G), 1)
    b = b_ref[...]

    @pl.when(pl.program_id(0) == 0)
    def _():
        s_ref[...] = jnp.zeros_like(s_ref)
        e_ref[...] = jnp.zeros_like(e_ref)

    s_ref[...] += jnp.sum((b < gi).astype(_i32), axis=0, keepdims=True)
    e_ref[...] += jnp.sum((b <= gi).astype(_i32), axis=0, keepdims=True)


def _gseg_edge_body(s_ref, p_ref, st_ref, en_ref, init_ref, o_ref):
    s = s_ref[...]
    m = jnp.logical_and(s >= st_ref[...], s < en_ref[...]).astype(_f32)
    sm = jnp.sum(m * p_ref[...], axis=0, keepdims=True)

    @pl.when(pl.program_id(0) == 0)
    def _():
        o_ref[...] = init_ref[...]

    o_ref[...] += sm


def _gseg_body(b_ref, p_ref, init_ref, o_ref):
    blk = b_ref.shape[0]
    gi = lax.broadcasted_iota(_i32, (blk, G), 1)
    m = (b_ref[...] == gi).astype(_f32) * p_ref[...]
    s = jnp.sum(m, axis=0, keepdims=True)

    @pl.when(pl.program_id(0) == 0)
    def _():
        o_ref[...] = init_ref[...]

    o_ref[...] += s


_MESH = plsc.VectorSubcoreMesh(core_axis_name="c", subcore_axis_name="s")


def _sc_gather_pair(tab_a, tab_b, sender, receiver):
    @functools.partial(
        pl.kernel,
        out_type=(jax.ShapeDtypeStruct((E, H), _f32),
                  jax.ShapeDtypeStruct((E, H), _f32)),
        mesh=_MESH,
        scratch_types=(pltpu.VMEM((EPW,), _i32), pltpu.VMEM((EPW,), _i32),
                       pltpu.VMEM((CH, H), _f32), pltpu.VMEM((CH, H), _f32),
                       pltpu.SemaphoreType.DMA, pltpu.SemaphoreType.DMA),
    )
    def k(ta, tb, si, ri, oa, ob, siv, riv, ra, rb, sa, sb):
        wid = lax.axis_index("s") * NC + lax.axis_index("c")
        base = wid * EPW
        pltpu.sync_copy(si.at[pl.ds(base, EPW)], siv)
        pltpu.sync_copy(ri.at[pl.ds(base, EPW)], riv)

        def body(i, carry):
            off = i * CH
            ca = pltpu.async_copy(ta.at[siv.at[pl.ds(off, CH)]], ra, sa)
            cb = pltpu.async_copy(tb.at[riv.at[pl.ds(off, CH)]], rb, sb)
            ca.wait()
            cb.wait()
            pltpu.sync_copy(ra, oa.at[pl.ds(base + off, CH)])
            pltpu.sync_copy(rb, ob.at[pl.ds(base + off, CH)])
            return carry

        lax.fori_loop(0, NCH, body, 0)

    return k(tab_a, tab_b, sender, receiver)


def _sc_segsum(values, ridx2, zeros_rows):
    @functools.partial(
        pl.kernel,
        out_type=jax.ShapeDtypeStruct((NC, N, H), _f32),
        mesh=_MESH,
        scratch_types=(pltpu.VMEM((CH, H), _f32),
                       pltpu.VMEM((NCH, CH), _i32),
                       pltpu.VMEM_SHARED((N, H), _f32)),
    )
    def k(vals_h, idx_h, zeros_h, out_h, rows_v, idx_v, accum):
        cid = lax.axis_index("c")
        sid = lax.axis_index("s")
        wid = sid * NC + cid
        base = wid * EPW
        pltpu.sync_copy(idx_h.at[wid], idx_v)
        pltpu.sync_copy(zeros_h, accum.at[pl.ds(sid * NPS, NPS)])

        @pl.when(sid == 0)
        def _():
            pltpu.sync_copy(zeros_h.at[pl.ds(0, NTAIL)],
                            accum.at[pl.ds(NS * NPS, NTAIL)])

        plsc.subcore_barrier()

        def body(i, carry):
            pltpu.sync_copy(vals_h.at[pl.ds(base + i * CH, CH)], rows_v)
            pltpu.sync_copy(rows_v, accum.at[idx_v.at[i]], add=True)
            return carry

        lax.fori_loop(0, NCH, body, 0)
        plsc.subcore_barrier()
        pltpu.sync_copy(accum.at[pl.ds(sid * NPS, NPS)],
                        out_h.at[cid, pl.ds(sid * NPS, NPS)])

        @pl.when(sid == 0)
        def _():
            pltpu.sync_copy(accum.at[pl.ds(NS * NPS, NTAIL)],
                            out_h.at[cid, pl.ds(NS * NPS, NTAIL)])

    return k(values, ridx2, zeros_rows)


def _espec(w=H):
    return pl.BlockSpec((EBLK, w), lambda i: (i, 0))


_WSPEC = pl.BlockSpec((H, H), lambda i: (0, 0))
_BSPEC = pl.BlockSpec((1, H), lambda i: (0, 0))


def _row(b):
    return b.reshape(1, -1)


def kernel(x_nodes, x_edges, edge_index, batch, pbc, params):
    p = params
    sender = edge_index[0]
    receiver = edge_index[1]

    (wne1, bne1), (wne2, bne2) = p["embed_nodes"]
    (wee1, bee1), (wee2, bee2) = p["embed_edges"]
    layers = []
    for lp in p["layers"]:
        (w1, b1), (w2, b2) = lp["edge_net"]
        (w3, b3), (w4, b4) = lp["node_net"]
        layers.append(dict(
            w1a=w1[0:H], w1b=w1[H:2 * H], w1c=w1[2 * H:3 * H],
            b1=_row(b1), w2=w2, b2=_row(b2),
            w3a=w3[0:H], w3b=w3[H:2 * H], b3=_row(b3), w4=w4, b4=_row(b4)))
    (wnr1, bnr1), (wnr2, bnr2) = p["node_readout"]
    (wer1, ber1), (wer2, ber2) = p["edge_readout"]
    wnr2p = jnp.pad(wnr2, ((0, 0), (0, H - wnr2.shape[1])))
    bnr2p = jnp.pad(_row(bnr2), ((0, 0), (0, H - bnr2.shape[0])))
    wer2p = jnp.pad(wer2, ((0, 0), (0, 16 - wer2.shape[1])))
    ber2p = jnp.pad(_row(ber2), ((0, 0), (0, 16 - ber2.shape[0])))

    ridx2 = receiver.reshape(NW, NCH, CH)
    zeros_rows = jnp.zeros((NPS, H), _f32)
    gzero = jnp.zeros((1, G), _f32)

    xn = pl.pallas_call(
        functools.partial(_mlp2_body, act_last=False),
        out_shape=jax.ShapeDtypeStruct((N, H), _f32),
    )(x_nodes, wne1, _row(bne1), wne2, _row(bne2))


    xe = None
    for li, lw in enumerate(layers):
        a_tab, b_tab = pl.pallas_call(
            _ab_body,
            out_shape=(jax.ShapeDtypeStruct((N, H), _f32),
                       jax.ShapeDtypeStruct((N, H), _f32)),
        )(xn, lw["w1a"], lw["w1b"])
        gs, gr = _sc_gather_pair(a_tab, b_tab, sender, receiver)

        if li == 0:
            xe = pl.pallas_call(
                _edge1_body,
                grid=(NEB,),
                in_specs=[pl.BlockSpec((EBLK, 16), lambda i: (i, 0)),
                          _espec(), _espec(),
                          pl.BlockSpec((16, H), lambda i: (0, 0)), _BSPEC,
                          _WSPEC, _BSPEC, _WSPEC, _BSPEC, _WSPEC, _BSPEC],
                out_specs=_espec(),
                out_shape=jax.ShapeDtypeStruct((E, H), _f32),
            )(x_edges, gs, gr, wee1, _row(bee1), wee2, _row(bee2),
              lw["w1c"], lw["b1"], lw["w2"], lw["b2"])
        elif li < 3:
            xe = pl.pallas_call(
                _edge_mid_body,
                grid=(NEB,),
                in_specs=[_espec(), _espec(), _espec(),
                          _WSPEC, _BSPEC, _WSPEC, _BSPEC],
                out_specs=_espec(),
                out_shape=jax.ShapeDtypeStruct((E, H), _f32),
            )(xe, gs, gr, lw["w1c"], lw["b1"], lw["w2"], lw["b2"])
        else:
            xe, xe_out_p, epur = pl.pallas_call(
                _edge4_body,
                grid=(NEB,),
                in_specs=[_espec(), _espec(), _espec(),
                          _WSPEC, _BSPEC, _WSPEC, _BSPEC,
                          _WSPEC, _BSPEC,
                          pl.BlockSpec((H, 16), lambda i: (0, 0)),
                          pl.BlockSpec((1, 16), lambda i: (0, 0))],
                out_specs=[_espec(), _espec(16), _espec(1)],
                out_shape=(jax.ShapeDtypeStruct((E, H), _f32),
                           jax.ShapeDtypeStruct((E, 16), _f32),
                           jax.ShapeDtypeStruct((E, 1), _f32)),
            )(xe, gs, gr, lw["w1c"], lw["b1"], lw["w2"], lw["b2"],
              wer1, _row(ber1), wer2p, ber2p)

        parts = _sc_segsum(xe, ridx2, zeros_rows)

        xn = pl.pallas_call(
            _node_body,
            out_shape=jax.ShapeDtypeStruct((N, H), _f32),
        )(xn, parts[0], parts[1], lw["w3a"], lw["w3b"], lw["b3"],
          lw["w4"], lw["b4"])

    xn_out_p, npur = pl.pallas_call(
        _nro_body,
        out_shape=(jax.ShapeDtypeStruct((N, H), _f32),
                   jax.ShapeDtypeStruct((N, 1), _f32)),
    )(xn, wnr1, _row(bnr1), wnr2p, bnr2p)

    NB = 2000
    gspec = pl.BlockSpec((1, G), lambda i: (0, 0))
    starts, ends = pl.pallas_call(
        _starts_body,
        grid=(N // NB,),
        in_specs=[pl.BlockSpec((NB, 1), lambda i: (i, 0))],
        out_specs=[gspec, gspec],
        out_shape=(jax.ShapeDtypeStruct((1, G), _i32),
                   jax.ShapeDtypeStruct((1, G), _i32)),
    )(batch.reshape(N, 1))
    ge = pl.pallas_call(
        _gseg_edge_body,
        grid=(NEB,),
        in_specs=[_espec(1), _espec(1), gspec, gspec, gspec],
        out_specs=gspec,
        out_shape=jax.ShapeDtypeStruct((1, G), _f32),
    )(sender.reshape(E, 1), epur, starts, ends, gzero)
    xg = pl.pallas_call(
        _gseg_body,
        grid=(N // NB,),
        in_specs=[pl.BlockSpec((NB, 1), lambda i: (i, 0)),
                  pl.BlockSpec((NB, 1), lambda i: (i, 0)), gspec],
        out_specs=gspec,
        out_shape=jax.ShapeDtypeStruct((1, G), _f32),
    )(batch.reshape(N, 1), npur, ge)

    return (xn_out_p[:, :3], xe_out_p[:, :15], xg.reshape(G))


# speedup vs baseline: 3.7663x; 1.1459x over previous
import functools

import jax
import jax.numpy as jnp
from jax import lax
from jax.experimental import pallas as pl
from jax.experimental.pallas import tpu as pltpu
from jax.experimental.pallas import tpu_sc as plsc

N = 10000
E = 320000
H = 128
G = 64
NC, NS = 2, 16
NW = NC * NS
EPW = E // NW
CH = 80
NCH = EPW // CH
NPS = 624
NTAIL = N - NS * NPS
EBLK = 8000
NEB = E // EBLK

_f32 = jnp.float32
_i32 = jnp.int32


def _silu(x):
    return x * (1.0 / (1.0 + jnp.exp(-x)))


def _mlp2_body(x_ref, w1_ref, b1_ref, w2_ref, b2_ref, o_ref, *, act_last):
    h = _silu(jnp.dot(x_ref[...], w1_ref[...], preferred_element_type=_f32)
              + b1_ref[...])
    o = jnp.dot(h, w2_ref[...], preferred_element_type=_f32) + b2_ref[...]
    o_ref[...] = _silu(o) if act_last else o


def _ab_body(x_ref, wa_ref, wb_ref, a_ref, b_ref):
    x = x_ref[...]
    a_ref[...] = jnp.dot(x, wa_ref[...], preferred_element_type=_f32)
    b_ref[...] = jnp.dot(x, wb_ref[...], preferred_element_type=_f32)


def _edge1_body(xr_ref, gs_ref, gr_ref, we1, be1, we2, be2, w1c, b1, w2, b2,
                o_ref):
    t = _silu(jnp.dot(xr_ref[...], we1[...], preferred_element_type=_f32)
              + be1[...])
    xe0 = jnp.dot(t, we2[...], preferred_element_type=_f32) + be2[...]
    h = _silu(jnp.dot(xe0, w1c[...], preferred_element_type=_f32)
              + gs_ref[...] + gr_ref[...] + b1[...])
    o = jnp.dot(h, w2[...], preferred_element_type=_f32) + b2[...]
    o_ref[...] = _silu(o)


def _edge_mid_body(xe_ref, gs_ref, gr_ref, w1c, b1, w2, b2, o_ref):
    h = _silu(jnp.dot(xe_ref[...], w1c[...], preferred_element_type=_f32)
              + gs_ref[...] + gr_ref[...] + b1[...])
    o = jnp.dot(h, w2[...], preferred_element_type=_f32) + b2[...]
    o_ref[...] = _silu(o)


def _edge4_body(xe_ref, gs_ref, gr_ref, w1c, b1, w2, b2, wr1, br1, wr2, br2,
                xe4_ref, xo_ref, ep_ref):
    h = _silu(jnp.dot(xe_ref[...], w1c[...], preferred_element_type=_f32)
              + gs_ref[...] + gr_ref[...] + b1[...])
    xe4 = _silu(jnp.dot(h, w2[...], preferred_element_type=_f32) + b2[...])
    xe4_ref[...] = xe4
    r = _silu(jnp.dot(xe4, wr1[...], preferred_element_type=_f32) + br1[...])
    o = jnp.dot(r, wr2[...], preferred_element_type=_f32) + br2[...]
    xo_ref[...] = o
    ep_ref[...] = (1.0 + jnp.sum(o * o, axis=1, keepdims=True)) * 0.25


def _node_body(xn_ref, p0_ref, p1_ref, w3a, w3b, b3, w4, b4, o_ref):
    aggr = p0_ref[...] + p1_ref[...]
    h = _silu(jnp.dot(xn_ref[...], w3a[...], preferred_element_type=_f32)
              + jnp.dot(aggr, w3b[...], preferred_element_type=_f32)
              + b3[...])
    o_ref[...] = jnp.dot(h, w4[...], preferred_element_type=_f32) + b4[...]


def _nro_body(xn_ref, wr1, br1, wr2, br2, xo_ref, np_ref):
    r = _silu(jnp.dot(xn_ref[...], wr1[...], preferred_element_type=_f32)
              + br1[...])
    o = jnp.dot(r, wr2[...], preferred_element_type=_f32) + br2[...]
    xo_ref[...] = o
    np_ref[...] = (1.0 + jnp.sum(o * o, axis=1, keepdims=True)) * 0.5


def _starts_body(b_ref, s_ref, e_ref):
    blk = b_ref.shape[0]
    gi = lax.broadcasted_iota(_i32, (blk, G), 1)
    b = b_ref[...]

    @pl.when(pl.program_id(0) == 0)
    def _():
        s_ref[...] = jnp.zeros_like(s_ref)
        e_ref[...] = jnp.zeros_like(e_ref)

    s_ref[...] += jnp.sum((b < gi).astype(_i32), axis=0, keepdims=True)
    e_ref[...] += jnp.sum((b <= gi).astype(_i32), axis=0, keepdims=True)


def _gseg_edge_body(s_ref, p_ref, st_ref, en_ref, init_ref, o_ref):
    s = s_ref[...]
    m = jnp.logical_and(s >= st_ref[...], s < en_ref[...]).astype(_f32)
    sm = jnp.sum(m * p_ref[...], axis=0, keepdims=True)

    @pl.when(pl.program_id(0) == 0)
    def _():
        o_ref[...] = init_ref[...]

    o_ref[...] += sm


def _gseg_body(b_ref, p_ref, init_ref, o_ref):
    blk = b_ref.shape[0]
    gi = lax.broadcasted_iota(_i32, (blk, G), 1)
    m = (b_ref[...] == gi).astype(_f32) * p_ref[...]
    s = jnp.sum(m, axis=0, keepdims=True)

    @pl.when(pl.program_id(0) == 0)
    def _():
        o_ref[...] = init_ref[...]

    o_ref[...] += s


_MESH = plsc.VectorSubcoreMesh(core_axis_name="c", subcore_axis_name="s")


def _sc_gather_pair(tab_a, tab_b, sender, receiver):
    @functools.partial(
        pl.kernel,
        out_type=(jax.ShapeDtypeStruct((E, H), _f32),
                  jax.ShapeDtypeStruct((E, H), _f32)),
        mesh=_MESH,
        scratch_types=(pltpu.VMEM((EPW,), _i32), pltpu.VMEM((EPW,), _i32),
                       pltpu.VMEM((CH, H), _f32), pltpu.VMEM((CH, H), _f32),
                       pltpu.VMEM((CH, H), _f32), pltpu.VMEM((CH, H), _f32),
                       pltpu.SemaphoreType.DMA, pltpu.SemaphoreType.DMA,
                       pltpu.SemaphoreType.DMA, pltpu.SemaphoreType.DMA),
    )
    def k(ta, tb, si, ri, oa, ob, siv, riv, ra0, rb0, ra1, rb1,
          g0, g1, w0, w1):
        wid = lax.axis_index("s") * NC + lax.axis_index("c")
        base = wid * EPW
        pltpu.sync_copy(si.at[pl.ds(base, EPW)], siv)
        pltpu.sync_copy(ri.at[pl.ds(base, EPW)], riv)

        def issue_gather(c, ra, rb, g):
            off = c * CH
            da = pltpu.async_copy(ta.at[siv.at[pl.ds(off, CH)]], ra, g)
            db = pltpu.async_copy(tb.at[riv.at[pl.ds(off, CH)]], rb, g)
            return da, db

        def issue_write(c, ra, rb, w):
            off = base + c * CH
            pltpu.async_copy(ra, oa.at[pl.ds(off, CH)], w)
            pltpu.async_copy(rb, ob.at[pl.ds(off, CH)], w)

        def wait_write(ra, rb, w):
            pltpu.make_async_copy(ra, oa.at[pl.ds(base, CH)], w).wait()
            pltpu.make_async_copy(rb, ob.at[pl.ds(base, CH)], w).wait()

        da0, db0 = issue_gather(0, ra0, rb0, g0)
        da1, db1 = issue_gather(1, ra1, rb1, g1)
        da0.wait(); db0.wait()
        issue_write(0, ra0, rb0, w0)
        da1.wait(); db1.wait()
        issue_write(1, ra1, rb1, w1)

        def body(i, carry):
            c = 2 + 2 * i
            wait_write(ra0, rb0, w0)
            da, db = issue_gather(c, ra0, rb0, g0)
            wait_write(ra1, rb1, w1)
            dc, dd = issue_gather(c + 1, ra1, rb1, g1)
            da.wait(); db.wait()
            issue_write(c, ra0, rb0, w0)
            dc.wait(); dd.wait()
            issue_write(c + 1, ra1, rb1, w1)
            return carry

        lax.fori_loop(0, (NCH - 2) // 2, body, 0)

        wait_write(ra0, rb0, w0)
        da, db = issue_gather(NCH - 1, ra0, rb0, g0)
        da.wait(); db.wait()
        pltpu.sync_copy(ra0, oa.at[pl.ds(base + (NCH - 1) * CH, CH)])
        pltpu.sync_copy(rb0, ob.at[pl.ds(base + (NCH - 1) * CH, CH)])
        wait_write(ra1, rb1, w1)

    return k(tab_a, tab_b, sender, receiver)


def _sc_segsum(values, ridx2, zeros_rows):
    @functools.partial(
        pl.kernel,
        out_type=jax.ShapeDtypeStruct((NC, N, H), _f32),
        mesh=_MESH,
        scratch_types=(pltpu.VMEM((CH, H), _f32), pltpu.VMEM((CH, H), _f32),
                       pltpu.VMEM((NCH, CH), _i32),
                       pltpu.VMEM_SHARED((N, H), _f32),
                       pltpu.SemaphoreType.DMA, pltpu.SemaphoreType.DMA,
                       pltpu.SemaphoreType.DMA, pltpu.SemaphoreType.DMA),
    )
    def k(vals_h, idx_h, zeros_h, out_h, v0, v1, idx_v, accum,
          l0, l1, s0, s1):
        cid = lax.axis_index("c")
        sid = lax.axis_index("s")
        wid = sid * NC + cid
        base = wid * EPW
        pltpu.sync_copy(idx_h.at[wid], idx_v)
        pltpu.sync_copy(zeros_h, accum.at[pl.ds(sid * NPS, NPS)])

        @pl.when(sid == 0)
        def _():
            pltpu.sync_copy(zeros_h.at[pl.ds(0, NTAIL)],
                            accum.at[pl.ds(NS * NPS, NTAIL)])

        plsc.subcore_barrier()

        def load(c, v, l):
            return pltpu.async_copy(vals_h.at[pl.ds(base + c * CH, CH)], v, l)

        def scatter(c, v, s):
            pltpu.async_copy(v, accum.at[idx_v.at[c]], s, add=True)

        def wait_scatter(v, s):
            pltpu.make_async_copy(v, accum.at[pl.ds(0, CH)], s).wait()

        dl0 = load(0, v0, l0)
        dl1 = load(1, v1, l1)
        dl0.wait()
        scatter(0, v0, s0)
        dl1.wait()
        scatter(1, v1, s1)

        def body(i, carry):
            c = 2 + 2 * i
            wait_scatter(v0, s0)
            da = load(c, v0, l0)
            wait_scatter(v1, s1)
            db = load(c + 1, v1, l1)
            da.wait()
            scatter(c, v0, s0)
            db.wait()
            scatter(c + 1, v1, s1)
            return carry

        lax.fori_loop(0, (NCH - 2) // 2, body, 0)

        wait_scatter(v0, s0)
        da = load(NCH - 1, v0, l0)
        da.wait()
        pltpu.sync_copy(v0, accum.at[idx_v.at[NCH - 1]], add=True)
        wait_scatter(v1, s1)
        plsc.subcore_barrier()
        pltpu.sync_copy(accum.at[pl.ds(sid * NPS, NPS)],
                        out_h.at[cid, pl.ds(sid * NPS, NPS)])

        @pl.when(sid == 0)
        def _():
            pltpu.sync_copy(accum.at[pl.ds(NS * NPS, NTAIL)],
                            out_h.at[cid, pl.ds(NS * NPS, NTAIL)])

    return k(values, ridx2, zeros_rows)


def _espec(w=H):
    return pl.BlockSpec((EBLK, w), lambda i: (i, 0))


_WSPEC = pl.BlockSpec((H, H), lambda i: (0, 0))
_BSPEC = pl.BlockSpec((1, H), lambda i: (0, 0))


def _row(b):
    return b.reshape(1, -1)


def kernel(x_nodes, x_edges, edge_index, batch, pbc, params):
    p = params
    sender = edge_index[0]
    receiver = edge_index[1]

    (wne1, bne1), (wne2, bne2) = p["embed_nodes"]
    (wee1, bee1), (wee2, bee2) = p["embed_edges"]
    layers = []
    for lp in p["layers"]:
        (w1, b1), (w2, b2) = lp["edge_net"]
        (w3, b3), (w4, b4) = lp["node_net"]
        layers.append(dict(
            w1a=w1[0:H], w1b=w1[H:2 * H], w1c=w1[2 * H:3 * H],
            b1=_row(b1), w2=w2, b2=_row(b2),
            w3a=w3[0:H], w3b=w3[H:2 * H], b3=_row(b3), w4=w4, b4=_row(b4)))
    (wnr1, bnr1), (wnr2, bnr2) = p["node_readout"]
    (wer1, ber1), (wer2, ber2) = p["edge_readout"]
    wnr2p = jnp.pad(wnr2, ((0, 0), (0, H - wnr2.shape[1])))
    bnr2p = jnp.pad(_row(bnr2), ((0, 0), (0, H - bnr2.shape[0])))
    wer2p = jnp.pad(wer2, ((0, 0), (0, 16 - wer2.shape[1])))
    ber2p = jnp.pad(_row(ber2), ((0, 0), (0, 16 - ber2.shape[0])))

    ridx2 = receiver.reshape(NW, NCH, CH)
    zeros_rows = jnp.zeros((NPS, H), _f32)
    gzero = jnp.zeros((1, G), _f32)

    xn = pl.pallas_call(
        functools.partial(_mlp2_body, act_last=False),
        out_shape=jax.ShapeDtypeStruct((N, H), _f32),
    )(x_nodes, wne1, _row(bne1), wne2, _row(bne2))


    xe = None
    for li, lw in enumerate(layers):
        a_tab, b_tab = pl.pallas_call(
            _ab_body,
            out_shape=(jax.ShapeDtypeStruct((N, H), _f32),
                       jax.ShapeDtypeStruct((N, H), _f32)),
        )(xn, lw["w1a"], lw["w1b"])
        gs, gr = _sc_gather_pair(a_tab, b_tab, sender, receiver)

        if li == 0:
            xe = pl.pallas_call(
                _edge1_body,
                grid=(NEB,),
                in_specs=[pl.BlockSpec((EBLK, 16), lambda i: (i, 0)),
                          _espec(), _espec(),
                          pl.BlockSpec((16, H), lambda i: (0, 0)), _BSPEC,
                          _WSPEC, _BSPEC, _WSPEC, _BSPEC, _WSPEC, _BSPEC],
                out_specs=_espec(),
                out_shape=jax.ShapeDtypeStruct((E, H), _f32),
            )(x_edges, gs, gr, wee1, _row(bee1), wee2, _row(bee2),
              lw["w1c"], lw["b1"], lw["w2"], lw["b2"])
        elif li < 3:
            xe = pl.pallas_call(
                _edge_mid_body,
                grid=(NEB,),
                in_specs=[_espec(), _espec(), _espec(),
                          _WSPEC, _BSPEC, _WSPEC, _BSPEC],
                out_specs=_espec(),
                out_shape=jax.ShapeDtypeStruct((E, H), _f32),
            )(xe, gs, gr, lw["w1c"], lw["b1"], lw["w2"], lw["b2"])
        else:
            xe, xe_out_p, epur = pl.pallas_call(
                _edge4_body,
                grid=(NEB,),
                in_specs=[_espec(), _espec(), _espec(),
                          _WSPEC, _BSPEC, _WSPEC, _BSPEC,
                          _WSPEC, _BSPEC,
                          pl.BlockSpec((H, 16), lambda i: (0, 0)),
                          pl.BlockSpec((1, 16), lambda i: (0, 0))],
                out_specs=[_espec(), _espec(16), _espec(1)],
                out_shape=(jax.ShapeDtypeStruct((E, H), _f32),
                           jax.ShapeDtypeStruct((E, 16), _f32),
                           jax.ShapeDtypeStruct((E, 1), _f32)),
            )(xe, gs, gr, lw["w1c"], lw["b1"], lw["w2"], lw["b2"],
              wer1, _row(ber1), wer2p, ber2p)

        parts = _sc_segsum(xe, ridx2, zeros_rows)

        xn = pl.pallas_call(
            _node_body,
            out_shape=jax.ShapeDtypeStruct((N, H), _f32),
        )(xn, parts[0], parts[1], lw["w3a"], lw["w3b"], lw["b3"],
          lw["w4"], lw["b4"])

    xn_out_p, npur = pl.pallas_call(
        _nro_body,
        out_shape=(jax.ShapeDtypeStruct((N, H), _f32),
                   jax.ShapeDtypeStruct((N, 1), _f32)),
    )(xn, wnr1, _row(bnr1), wnr2p, bnr2p)

    NB = 2000
    gspec = pl.BlockSpec((1, G), lambda i: (0, 0))
    starts, ends = pl.pallas_call(
        _starts_body,
        grid=(N // NB,),
        in_specs=[pl.BlockSpec((NB, 1), lambda i: (i, 0))],
        out_specs=[gspec, gspec],
        out_shape=(jax.ShapeDtypeStruct((1, G), _i32),
                   jax.ShapeDtypeStruct((1, G), _i32)),
    )(batch.reshape(N, 1))
    ge = pl.pallas_call(
        _gseg_edge_body,
        grid=(NEB,),
        in_specs=[_espec(1), _espec(1), gspec, gspec, gspec],
        out_specs=gspec,
        out_shape=jax.ShapeDtypeStruct((1, G), _f32),
    )(sender.reshape(E, 1), epur, starts, ends, gzero)
    xg = pl.pallas_call(
        _gseg_body,
        grid=(N // NB,),
        in_specs=[pl.BlockSpec((NB, 1), lambda i: (i, 0)),
                  pl.BlockSpec((NB, 1), lambda i: (i, 0)), gspec],
        out_specs=gspec,
        out_shape=jax.ShapeDtypeStruct((1, G), _f32),
    )(batch.reshape(N, 1), npur, ge)

    return (xn_out_p[:, :3], xe_out_p[:, :15], xg.reshape(G))


# speedup vs baseline: 3.7890x; 1.0060x over previous
import functools

import jax
import jax.numpy as jnp
from jax import lax
from jax.experimental import pallas as pl
from jax.experimental.pallas import tpu as pltpu
from jax.experimental.pallas import tpu_sc as plsc

N = 10000
E = 320000
H = 128
G = 64
NC, NS = 2, 16
NW = NC * NS
EPW = E // NW
CH = 80
NCH = EPW // CH
NPS = 624
NTAIL = N - NS * NPS
EBLK = 8000
NEB = E // EBLK

_f32 = jnp.float32
_i32 = jnp.int32
_bf16 = jnp.bfloat16


def _silu(x):
    return x * lax.logistic(x)


def _dot(x, w):
    return jnp.dot(x, w, preferred_element_type=_f32)


def _mlp2_body(x_ref, w1_ref, b1_ref, w2_ref, b2_ref, o_ref, *, act_last):
    h = _silu(_dot(x_ref[...], w1_ref[...])
              + b1_ref[...])
    o = _dot(h, w2_ref[...]) + b2_ref[...]
    o_ref[...] = _silu(o) if act_last else o


def _edge1_body(xr_ref, gs_ref, gr_ref, we1, be1, we2, be2, w1, b1, w2, b2,
                o_ref):
    t = _silu(_dot(xr_ref[...], we1[...])
              + be1[...])
    xe0 = _dot(t, we2[...]) + be2[...]
    state = jnp.concatenate([gs_ref[...], gr_ref[...], xe0], axis=1)
    h = _silu(_dot(state, w1[...]) + b1[...])
    o = _dot(h, w2[...]) + b2[...]
    o_ref[...] = _silu(o)


def _edge_mid_body(xe_ref, gs_ref, gr_ref, w1, b1, w2, b2, o_ref):
    state = jnp.concatenate([gs_ref[...], gr_ref[...], xe_ref[...]], axis=1)
    h = _silu(_dot(state, w1[...]) + b1[...])
    o = _dot(h, w2[...]) + b2[...]
    o_ref[...] = _silu(o)


def _edge4_body(xe_ref, gs_ref, gr_ref, w1, b1, w2, b2, wr1, br1, wr2, br2,
                xe4_ref, xo_ref, ep_ref):
    state = jnp.concatenate([gs_ref[...], gr_ref[...], xe_ref[...]], axis=1)
    h = _silu(_dot(state, w1[...]) + b1[...])
    xe4 = _silu(_dot(h, w2[...]) + b2[...])
    xe4_ref[...] = xe4
    r = _silu(_dot(xe4, wr1[...]) + br1[...])
    o = _dot(r, wr2[...]) + br2[...]
    xo_ref[...] = o
    ep_ref[...] = (1.0 + jnp.sum(o * o, axis=1, keepdims=True)) * 0.25


def _node_body(xn_ref, p0_ref, p1_ref, w3, b3, w4, b4, o_ref):
    x = jnp.concatenate([xn_ref[...], p0_ref[...] + p1_ref[...]], axis=1)
    h = _silu(_dot(x, w3[...]) + b3[...])
    o_ref[...] = _dot(h, w4[...]) + b4[...]


def _nro_body(xn_ref, wr1, br1, wr2, br2, xo_ref, np_ref):
    r = _silu(_dot(xn_ref[...], wr1[...])
              + br1[...])
    o = _dot(r, wr2[...]) + br2[...]
    xo_ref[...] = o
    np_ref[...] = (1.0 + jnp.sum(o * o, axis=1, keepdims=True)) * 0.5


def _starts_body(b_ref, s_ref, e_ref):
    blk = b_ref.shape[0]
    gi = lax.broadcasted_iota(_i32, (blk, G), 1)
    b = b_ref[...]

    @pl.when(pl.program_id(0) == 0)
    def _():
        s_ref[...] = jnp.zeros_like(s_ref)
        e_ref[...] = jnp.zeros_like(e_ref)

    s_ref[...] += jnp.sum((b < gi).astype(_i32), axis=0, keepdims=True)
    e_ref[...] += jnp.sum((b <= gi).astype(_i32), axis=0, keepdims=True)


def _gseg_edge_body(s_ref, p_ref, st_ref, en_ref, init_ref, o_ref):
    s = s_ref[...]
    m = jnp.logical_and(s >= st_ref[...], s < en_ref[...]).astype(_f32)
    sm = jnp.sum(m * p_ref[...], axis=0, keepdims=True)

    @pl.when(pl.program_id(0) == 0)
    def _():
        o_ref[...] = init_ref[...]

    o_ref[...] += sm


def _gseg_body(b_ref, p_ref, init_ref, o_ref):
    blk = b_ref.shape[0]
    gi = lax.broadcasted_iota(_i32, (blk, G), 1)
    m = (b_ref[...] == gi).astype(_f32) * p_ref[...]
    s = jnp.sum(m, axis=0, keepdims=True)

    @pl.when(pl.program_id(0) == 0)
    def _():
        o_ref[...] = init_ref[...]

    o_ref[...] += s


_MESH = plsc.VectorSubcoreMesh(core_axis_name="c", subcore_axis_name="s")


def _sc_gather_pair(tab, sender, receiver):
    @functools.partial(
        pl.kernel,
        out_type=(jax.ShapeDtypeStruct((E, H), _f32),
                  jax.ShapeDtypeStruct((E, H), _f32)),
        mesh=_MESH,
        scratch_types=(pltpu.VMEM((EPW,), _i32), pltpu.VMEM((EPW,), _i32),
                       pltpu.VMEM((CH, H), _f32), pltpu.VMEM((CH, H), _f32),
                       pltpu.VMEM((CH, H), _f32), pltpu.VMEM((CH, H), _f32),
                       pltpu.SemaphoreType.DMA, pltpu.SemaphoreType.DMA,
                       pltpu.SemaphoreType.DMA, pltpu.SemaphoreType.DMA),
    )
    def k(ta, si, ri, oa, ob, siv, riv, ra0, rb0, ra1, rb1, g0, g1, w0, w1):
        wid = lax.axis_index("s") * NC + lax.axis_index("c")
        base = wid * EPW
        pltpu.sync_copy(si.at[pl.ds(base, EPW)], siv)
        pltpu.sync_copy(ri.at[pl.ds(base, EPW)], riv)

        def issue_gather(c, ra, rb, g):
            off = c * CH
            da = pltpu.async_copy(ta.at[siv.at[pl.ds(off, CH)]], ra, g)
            db = pltpu.async_copy(ta.at[riv.at[pl.ds(off, CH)]], rb, g)
            return da, db

        def issue_write(c, ra, rb, w):
            off = base + c * CH
            pltpu.async_copy(ra, oa.at[pl.ds(off, CH)], w)
            pltpu.async_copy(rb, ob.at[pl.ds(off, CH)], w)

        def wait_write(ra, rb, w):
            pltpu.make_async_copy(ra, oa.at[pl.ds(base, CH)], w).wait()
            pltpu.make_async_copy(rb, ob.at[pl.ds(base, CH)], w).wait()

        da0, db0 = issue_gather(0, ra0, rb0, g0)
        da1, db1 = issue_gather(1, ra1, rb1, g1)
        da0.wait(); db0.wait()
        issue_write(0, ra0, rb0, w0)
        da1.wait(); db1.wait()
        issue_write(1, ra1, rb1, w1)

        def body(i, carry):
            c = 2 + 2 * i
            wait_write(ra0, rb0, w0)
            da, db = issue_gather(c, ra0, rb0, g0)
            wait_write(ra1, rb1, w1)
            dc, dd = issue_gather(c + 1, ra1, rb1, g1)
            da.wait(); db.wait()
            issue_write(c, ra0, rb0, w0)
            dc.wait(); dd.wait()
            issue_write(c + 1, ra1, rb1, w1)
            return carry

        lax.fori_loop(0, (NCH - 2) // 2, body, 0)

        wait_write(ra0, rb0, w0)
        da, db = issue_gather(NCH - 1, ra0, rb0, g0)
        da.wait(); db.wait()
        pltpu.sync_copy(ra0, oa.at[pl.ds(base + (NCH - 1) * CH, CH)])
        pltpu.sync_copy(rb0, ob.at[pl.ds(base + (NCH - 1) * CH, CH)])
        wait_write(ra1, rb1, w1)

    return k(tab, sender, receiver)


def _sc_segsum(values, ridx2, zeros_rows):
    @functools.partial(
        pl.kernel,
        out_type=jax.ShapeDtypeStruct((NC, N, H), _f32),
        mesh=_MESH,
        scratch_types=(pltpu.VMEM((CH, H), _f32), pltpu.VMEM((CH, H), _f32),
                       pltpu.VMEM((NCH, CH), _i32),
                       pltpu.VMEM_SHARED((N, H), _f32),
                       pltpu.SemaphoreType.DMA, pltpu.SemaphoreType.DMA,
                       pltpu.SemaphoreType.DMA, pltpu.SemaphoreType.DMA),
    )
    def k(vals_h, idx_h, zeros_h, out_h, v0, v1, idx_v, accum,
          l0, l1, s0, s1):
        cid = lax.axis_index("c")
        sid = lax.axis_index("s")
        wid = sid * NC + cid
        base = wid * EPW
        pltpu.sync_copy(idx_h.at[wid], idx_v)
        pltpu.sync_copy(zeros_h, accum.at[pl.ds(sid * NPS, NPS)])

        @pl.when(sid == 0)
        def _():
            pltpu.sync_copy(zeros_h.at[pl.ds(0, NTAIL)],
                            accum.at[pl.ds(NS * NPS, NTAIL)])

        plsc.subcore_barrier()

        def load(c, v, l):
            return pltpu.async_copy(vals_h.at[pl.ds(base + c * CH, CH)], v, l)

        def scatter(c, v, s):
            pltpu.async_copy(v, accum.at[idx_v.at[c]], s, add=True)

        def wait_scatter(v, s):
            pltpu.make_async_copy(v, accum.at[pl.ds(0, CH)], s).wait()

        dl0 = load(0, v0, l0)
        dl1 = load(1, v1, l1)
        dl0.wait()
        scatter(0, v0, s0)
        dl1.wait()
        scatter(1, v1, s1)

        def body(i, carry):
            c = 2 + 2 * i
            wait_scatter(v0, s0)
            da = load(c, v0, l0)
            wait_scatter(v1, s1)
            db = load(c + 1, v1, l1)
            da.wait()
            scatter(c, v0, s0)
            db.wait()
            scatter(c + 1, v1, s1)
            return carry

        lax.fori_loop(0, (NCH - 2) // 2, body, 0)

        wait_scatter(v0, s0)
        da = load(NCH - 1, v0, l0)
        da.wait()
        pltpu.sync_copy(v0, accum.at[idx_v.at[NCH - 1]], add=True)
        wait_scatter(v1, s1)
        plsc.subcore_barrier()
        pltpu.sync_copy(accum.at[pl.ds(sid * NPS, NPS)],
                        out_h.at[cid, pl.ds(sid * NPS, NPS)])

        @pl.when(sid == 0)
        def _():
            pltpu.sync_copy(accum.at[pl.ds(NS * NPS, NTAIL)],
                            out_h.at[cid, pl.ds(NS * NPS, NTAIL)])

    return k(values, ridx2, zeros_rows)


def _espec(w=H):
    return pl.BlockSpec((EBLK, w), lambda i: (i, 0))


_WSPEC = pl.BlockSpec((H, H), lambda i: (0, 0))
_W1SPEC = pl.BlockSpec((3 * H, H), lambda i: (0, 0))
_BSPEC = pl.BlockSpec((1, H), lambda i: (0, 0))


def _row(b):
    return b.reshape(1, -1)


def kernel(x_nodes, x_edges, edge_index, batch, pbc, params):
    p = params
    sender = edge_index[0]
    receiver = edge_index[1]

    (wne1, bne1), (wne2, bne2) = p["embed_nodes"]
    (wee1, bee1), (wee2, bee2) = p["embed_edges"]
    layers = []
    for lp in p["layers"]:
        (w1, b1), (w2, b2) = lp["edge_net"]
        (w3, b3), (w4, b4) = lp["node_net"]
        layers.append(dict(
            w1=w1, b1=_row(b1), w2=w2, b2=_row(b2),
            w3=w3, b3=_row(b3), w4=w4, b4=_row(b4)))
    (wnr1, bnr1), (wnr2, bnr2) = p["node_readout"]
    (wer1, ber1), (wer2, ber2) = p["edge_readout"]
    wnr2p = jnp.pad(wnr2, ((0, 0), (0, H - wnr2.shape[1])))
    bnr2p = jnp.pad(_row(bnr2), ((0, 0), (0, H - bnr2.shape[0])))
    wer2p = jnp.pad(wer2, ((0, 0), (0, 16 - wer2.shape[1])))
    ber2p = jnp.pad(_row(ber2), ((0, 0), (0, 16 - ber2.shape[0])))

    ridx2 = receiver.reshape(NW, NCH, CH)
    zeros_rows = jnp.zeros((NPS, H), _f32)
    gzero = jnp.zeros((1, G), _f32)

    xn = pl.pallas_call(
        functools.partial(_mlp2_body, act_last=False),
        out_shape=jax.ShapeDtypeStruct((N, H), _f32),
    )(x_nodes, wne1, _row(bne1), wne2, _row(bne2))


    xe = None
    for li, lw in enumerate(layers):
        gs, gr = _sc_gather_pair(xn, sender, receiver)

        if li == 0:
            xe = pl.pallas_call(
                _edge1_body,
                grid=(NEB,),
                in_specs=[pl.BlockSpec((EBLK, 16), lambda i: (i, 0)),
                          _espec(), _espec(),
                          pl.BlockSpec((16, H), lambda i: (0, 0)), _BSPEC,
                          _WSPEC, _BSPEC, _W1SPEC, _BSPEC, _WSPEC, _BSPEC],
                out_specs=_espec(),
                out_shape=jax.ShapeDtypeStruct((E, H), _f32),
            )(x_edges, gs, gr, wee1, _row(bee1), wee2, _row(bee2),
              lw["w1"], lw["b1"], lw["w2"], lw["b2"])
        elif li < 3:
            xe = pl.pallas_call(
                _edge_mid_body,
                grid=(NEB,),
                in_specs=[_espec(), _espec(), _espec(),
                          _W1SPEC, _BSPEC, _WSPEC, _BSPEC],
                out_specs=_espec(),
                out_shape=jax.ShapeDtypeStruct((E, H), _f32),
            )(xe, gs, gr, lw["w1"], lw["b1"], lw["w2"], lw["b2"])
        else:
            xe, xe_out_p, epur = pl.pallas_call(
                _edge4_body,
                grid=(NEB,),
                in_specs=[_espec(), _espec(), _espec(),
                          _W1SPEC, _BSPEC, _WSPEC, _BSPEC,
                          _WSPEC, _BSPEC,
                          pl.BlockSpec((H, 16), lambda i: (0, 0)),
                          pl.BlockSpec((1, 16), lambda i: (0, 0))],
                out_specs=[_espec(), _espec(16), _espec(1)],
                out_shape=(jax.ShapeDtypeStruct((E, H), _f32),
                           jax.ShapeDtypeStruct((E, 16), _f32),
                           jax.ShapeDtypeStruct((E, 1), _f32)),
            )(xe, gs, gr, lw["w1"], lw["b1"], lw["w2"], lw["b2"],
              wer1, _row(ber1), wer2p, ber2p)

        parts = _sc_segsum(xe, ridx2, zeros_rows)

        xn = pl.pallas_call(
            _node_body,
            out_shape=jax.ShapeDtypeStruct((N, H), _f32),
        )(xn, parts[0], parts[1], lw["w3"], lw["b3"], lw["w4"], lw["b4"])

    xn_out_p, npur = pl.pallas_call(
        _nro_body,
        out_shape=(jax.ShapeDtypeStruct((N, H), _f32),
                   jax.ShapeDtypeStruct((N, 1), _f32)),
    )(xn, wnr1, _row(bnr1), wnr2p, bnr2p)

    NB = 2000
    gspec = pl.BlockSpec((1, G), lambda i: (0, 0))
    starts, ends = pl.pallas_call(
        _starts_body,
        grid=(N // NB,),
        in_specs=[pl.BlockSpec((NB, 1), lambda i: (i, 0))],
        out_specs=[gspec, gspec],
        out_shape=(jax.ShapeDtypeStruct((1, G), _i32),
                   jax.ShapeDtypeStruct((1, G), _i32)),
    )(batch.reshape(N, 1))
    ge = pl.pallas_call(
        _gseg_edge_body,
        grid=(NEB,),
        in_specs=[_espec(1), _espec(1), gspec, gspec, gspec],
        out_specs=gspec,
        out_shape=jax.ShapeDtypeStruct((1, G), _f32),
    )(sender.reshape(E, 1), epur, starts, ends, gzero)
    xg = pl.pallas_call(
        _gseg_body,
        grid=(N // NB,),
        in_specs=[pl.BlockSpec((NB, 1), lambda i: (i, 0)),
                  pl.BlockSpec((NB, 1), lambda i: (i, 0)), gspec],
        out_specs=gspec,
        out_shape=jax.ShapeDtypeStruct((1, G), _f32),
    )(batch.reshape(N, 1), npur, ge)

    return (xn_out_p[:, :3], xe_out_p[:, :15], xg.reshape(G))
